# baseline (speedup 1.0000x reference)
import jax
import jax.numpy as jnp
from jax import lax
from jax.experimental import pallas as pl
from jax.experimental.pallas import tpu as pltpu
from jax.experimental.pallas import tpu_sc as plsc

N = 10000
E = 320000
HID = 128
OUT = 128

NC = 2
NS = 16
QW = HID // NC

CHUNK = 64
NCHUNK = 320
EPAD = NS * NCHUNK * CHUNK
KB = 16
NBLK = NCHUNK // KB
IBUF = 2
RING = 5
DEPTH = 2

NPAD = 10240
AGG_PER_TILE = NPAD // NS

NHALF = N // NC
CNT_SPAN = NHALF * 9
CNT_WORDS = 46080


def _tc1_body(x_ref, a_ref, w_ref, o_ref):
    xb = x_ref[...]
    hb = jnp.where(xb >= 0, xb, xb * a_ref[...])
    hb = lax.dot_general(hb, w_ref[...], (((1,), (1,)), ((), ())),
                         preferred_element_type=jnp.float32)
    for q in range(NC):
        o_ref[q] = hb[:, q * QW:(q + 1) * QW]


def _tc1(x, a_row, W):
    blk = N // 10
    return pl.pallas_call(
        _tc1_body,
        grid=(10,),
        in_specs=[
            pl.BlockSpec((blk, HID), lambda i: (i, 0)),
            pl.BlockSpec((1, HID), lambda i: (0, 0)),
            pl.BlockSpec((HID, HID), lambda i: (0, 0)),
        ],
        out_specs=pl.BlockSpec((NC, blk, QW), lambda i: (0, i, 0)),
        out_shape=jax.ShapeDtypeStruct((NC, N, QW), jnp.float32),
    )(x, a_row, W)


def _sc_body(h2_hbm, pack_hbm,
             agg_out, cnt_out,
             idx_v, rows_v, cnt_v, sem_i, sem_g, sem_s,
             agg_sh):
    c = lax.axis_index("c")
    s = lax.axis_index("s")

    z16 = jnp.zeros((16,), jnp.float32)
    ones = jnp.ones((16,), jnp.float32)
    cnt_lo = c * CNT_SPAN

    def zrow(i, carry):
        for g in range(QW // 16):
            rows_v[0, i, pl.ds(g * 16, 16)] = z16
        return carry
    lax.fori_loop(0, CHUNK, zrow, 0)

    def zcnt(i, carry):
        for g in range(8):
            cnt_v[pl.ds(i * 128 + g * 16, 16)] = z16
        return carry
    lax.fori_loop(0, CNT_WORDS // 128, zcnt, 0)

    for r in range(AGG_PER_TILE // CHUNK):
        pltpu.sync_copy(
            rows_v.at[0],
            agg_sh.at[pl.ds(s * AGG_PER_TILE + r * CHUNK, CHUNK)])
    plsc.subcore_barrier()

    def gather_desc(j):
        b = lax.rem(lax.div(j, KB), IBUF)
        return pltpu.make_async_copy(
            h2_hbm.at[c].at[idx_v.at[b, lax.rem(j, KB), 0]],
            rows_v.at[lax.rem(j, RING)], sem_g)

    def scatter_desc(j):
        b = lax.rem(lax.div(j, KB), IBUF)
        return pltpu.make_async_copy(
            rows_v.at[lax.rem(j, RING)],
            agg_sh.at[idx_v.at[b, lax.rem(j, KB), 1]], sem_s)

    def idx_desc(blk):
        return pltpu.make_async_copy(
            pack_hbm.at[s, pl.ds(blk * KB, KB)],
            idx_v.at[lax.rem(blk, IBUF)], sem_i)

    idx_desc(0).start()

    def body(j, carry):
        slot = lax.rem(j, KB)
        blk = lax.div(j, KB)

        @pl.when(j >= RING - DEPTH)
        def _():
            scatter_desc(j - (RING - DEPTH)).wait()

        @pl.when(slot == 0)
        def _():
            idx_desc(blk).wait()

            @pl.when(blk + 1 < NBLK)
            def _():
                idx_desc(blk + 1).start()

            for t in range(DEPTH):
                @pl.when(j + t < NCHUNK)
                def _():
                    gather_desc(j + t).start()

        @pl.when((slot + DEPTH < KB) & (j + DEPTH < NCHUNK))
        def _():
            gather_desc(j + DEPTH).start()

        gather_desc(j).wait()
        scatter_desc(j).start(add=True)

        b = lax.rem(blk, IBUF)
        for g in range(CHUNK // 16):
            sl = pl.ds(g * 16, 16)
            flat = (idx_v[b, slot, 1, sl] * 9
                    + idx_v[b, slot, 2, sl] * 3
                    + idx_v[b, slot, 3, sl]) - cnt_lo
            mask = (flat >= 0) & (flat < CNT_SPAN)
            flat = jnp.where(mask, flat, 0)
            plsc.addupdate_scatter(cnt_v, [flat], ones, mask=mask)

        return carry

    lax.fori_loop(0, NCHUNK, body, 0)

    for t in range(NCHUNK - (RING - DEPTH), NCHUNK):
        scatter_desc(t).wait()

    pltpu.sync_copy(cnt_v, cnt_out.at[c, s])
    plsc.subcore_barrier()

    pltpu.sync_copy(agg_sh.at[pl.ds(s * AGG_PER_TILE, AGG_PER_TILE)],
                    agg_out.at[c, pl.ds(s * AGG_PER_TILE, AGG_PER_TILE)])


_sc_call = pl.kernel(
    _sc_body,
    out_type=(
        jax.ShapeDtypeStruct((NC, NPAD, QW), jnp.float32),
        jax.ShapeDtypeStruct((NC, NS, CNT_WORDS), jnp.float32),
    ),
    mesh=plsc.VectorSubcoreMesh(core_axis_name="c", subcore_axis_name="s"),
    compiler_params=pltpu.CompilerParams(needs_layout_passes=False,
                                         use_tc_tiling_on_sc=False),
    scratch_types=[
        pltpu.VMEM((IBUF, KB, 4, CHUNK), jnp.int32),
        pltpu.VMEM((RING, CHUNK, QW), jnp.float32),
        pltpu.VMEM((CNT_WORDS,), jnp.float32),
        pltpu.SemaphoreType.DMA,
        pltpu.SemaphoreType.DMA,
        pltpu.SemaphoreType.DMA,
        pltpu.VMEM_SHARED((NPAD, QW), jnp.float32),
    ],
)


def _tcsum_body(c_ref, o_ref):
    o_ref[...] = jnp.sum(c_ref[...], axis=1)


def _tcsum(cnt_parts):
    blk = CNT_WORDS // 5
    return pl.pallas_call(
        _tcsum_body,
        grid=(5,),
        in_specs=[pl.BlockSpec((NC, NS, blk), lambda i: (0, 0, i))],
        out_specs=pl.BlockSpec((NC, blk), lambda i: (0, i)),
        out_shape=jax.ShapeDtypeStruct((NC, CNT_WORDS), jnp.float32),
    )(cnt_parts)


def _tc2_body(agg_ref, h_ref, cnt_ref, comb_ref, self_ref,
              w1_ref, b1_ref, w2_ref, b2_ref, o_ref):
    a = (jnp.concatenate([agg_ref[q] for q in range(NC)], axis=1)
         + jnp.concatenate([h_ref[q] for q in range(NC)], axis=1)
         + self_ref[...])
    a = a + lax.dot_general(cnt_ref[...], comb_ref[...],
                            (((1,), (0,)), ((), ())),
                            preferred_element_type=jnp.float32)
    z = lax.dot_general(a, w1_ref[...], (((1,), (1,)), ((), ())),
                        preferred_element_type=jnp.float32) + b1_ref[...]
    z = jnp.maximum(z, 0.0)
    o_ref[...] = lax.dot_general(z, w2_ref[...], (((1,), (1,)), ((), ())),
                                 preferred_element_type=jnp.float32) + b2_ref[...]


def _tc2(agg_parts, h2, cnt, comb, selfvec, W1, b1, W2, b2):
    blk = N // 10
    return pl.pallas_call(
        _tc2_body,
        grid=(10,),
        in_specs=[
            pl.BlockSpec((NC, blk, QW), lambda i: (0, i, 0)),
            pl.BlockSpec((NC, blk, QW), lambda i: (0, i, 0)),
            pl.BlockSpec((blk, 9), lambda i: (i, 0)),
            pl.BlockSpec((9, HID), lambda i: (0, 0)),
            pl.BlockSpec((1, HID), lambda i: (0, 0)),
            pl.BlockSpec((2 * HID, HID), lambda i: (0, 0)),
            pl.BlockSpec((1, 2 * HID), lambda i: (0, 0)),
            pl.BlockSpec((OUT, 2 * HID), lambda i: (0, 0)),
            pl.BlockSpec((1, OUT), lambda i: (0, 0)),
        ],
        out_specs=pl.BlockSpec((blk, OUT), lambda i: (i, 0)),
        out_shape=jax.ShapeDtypeStruct((N, OUT), jnp.float32),
    )(agg_parts, h2, cnt, comb, selfvec, W1, b1, W2, b2)


def kernel(x, edge_index, edge_attr, prelu_a, W, W1, b1, W2, b2, emb1, emb2):
    a_row = jnp.broadcast_to(prelu_a, (1, HID)).astype(jnp.float32)
    h2 = _tc1(x, a_row, W)

    pack = jnp.concatenate([edge_index, edge_attr.T], axis=0)
    npadrow = EPAD - E
    zpad = jnp.zeros((npadrow,), jnp.int32)
    pad_dst = N + (jnp.arange(npadrow, dtype=jnp.int32) % (NPAD - N))
    pack = jnp.concatenate(
        [pack, jnp.stack([zpad, pad_dst, zpad, zpad])], axis=1)
    pack = pack.reshape(4, NS, NCHUNK, CHUNK).transpose(1, 2, 0, 3)

    agg_parts, cnt_parts = _sc_call(h2, pack)

    csum = _tcsum(cnt_parts)
    cnt = jnp.concatenate(
        [csum[0, :CNT_SPAN], csum[1, :CNT_SPAN]]).reshape(N, 9)
    comb = (emb1[:3, None, :] + emb2[None, :3, :]).reshape(9, HID)
    selfvec = (emb1[4] + emb2[0]).reshape(1, HID)

    return _tc2(agg_parts, h2, cnt, comb, selfvec,
                W1, b1.reshape(1, -1), W2, b2.reshape(1, -1))


# speedup vs baseline: 1.0042x; 1.0042x over previous
import jax
import jax.numpy as jnp
from jax import lax
from jax.experimental import pallas as pl
from jax.experimental.pallas import tpu as pltpu
from jax.experimental.pallas import tpu_sc as plsc

N = 10000
E = 320000
HID = 128
OUT = 128

NC = 2
NS = 16
QW = HID // NC

CHUNK = 64
NCHUNK = 320
EPAD = NS * NCHUNK * CHUNK
KB = 16
NBLK = NCHUNK // KB
IBUF = 2
RING = 4
DEPTH = 2

NPAD = 10240
AGG_PER_TILE = NPAD // NS

NHALF = N // NC
CNT_SPAN = NHALF * 9
CNT_WORDS = 46080


def _tc1_body(x_ref, a_ref, w_ref, o_ref):
    xb = x_ref[...]
    hb = jnp.where(xb >= 0, xb, xb * a_ref[...])
    hb = lax.dot_general(hb, w_ref[...], (((1,), (1,)), ((), ())),
                         preferred_element_type=jnp.float32)
    for q in range(NC):
        o_ref[q] = hb[:, q * QW:(q + 1) * QW]


def _tc1(x, a_row, W):
    blk = N // 10
    return pl.pallas_call(
        _tc1_body,
        grid=(10,),
        in_specs=[
            pl.BlockSpec((blk, HID), lambda i: (i, 0)),
            pl.BlockSpec((1, HID), lambda i: (0, 0)),
            pl.BlockSpec((HID, HID), lambda i: (0, 0)),
        ],
        out_specs=pl.BlockSpec((NC, blk, QW), lambda i: (0, i, 0)),
        out_shape=jax.ShapeDtypeStruct((NC, N, QW), jnp.float32),
    )(x, a_row, W)


def _sc_body(h2_hbm, pack_hbm,
             agg_out, cnt_out,
             idx_v, rows_v, cnt_v, sem_i, sem_g, sem_s,
             agg_sh):
    c = lax.axis_index("c")
    s = lax.axis_index("s")

    z16 = jnp.zeros((16,), jnp.float32)
    ones = jnp.ones((16,), jnp.float32)
    cnt_lo = c * CNT_SPAN

    def zrow(i, carry):
        for g in range(QW // 16):
            rows_v[0, i, pl.ds(g * 16, 16)] = z16
        return carry
    lax.fori_loop(0, CHUNK, zrow, 0)

    def zcnt(i, carry):
        for g in range(8):
            cnt_v[pl.ds(i * 128 + g * 16, 16)] = z16
        return carry
    lax.fori_loop(0, CNT_WORDS // 128, zcnt, 0)

    for r in range(AGG_PER_TILE // CHUNK):
        pltpu.sync_copy(
            rows_v.at[0],
            agg_sh.at[pl.ds(s * AGG_PER_TILE + r * CHUNK, CHUNK)])
    plsc.subcore_barrier()

    def gather_desc(j):
        b = lax.rem(lax.div(j, KB), IBUF)
        return pltpu.make_async_copy(
            h2_hbm.at[c].at[idx_v.at[b, lax.rem(j, KB), 0]],
            rows_v.at[lax.rem(j, RING)], sem_g)

    def scatter_desc(j):
        b = lax.rem(lax.div(j, KB), IBUF)
        return pltpu.make_async_copy(
            rows_v.at[lax.rem(j, RING)],
            agg_sh.at[idx_v.at[b, lax.rem(j, KB), 1]], sem_s)

    def idx_desc(blk):
        return pltpu.make_async_copy(
            pack_hbm.at[s, pl.ds(blk * KB, KB)],
            idx_v.at[lax.rem(blk, IBUF)], sem_i)

    idx_desc(0).start()

    def body(j, carry):
        slot = lax.rem(j, KB)
        blk = lax.div(j, KB)

        @pl.when(j >= RING - DEPTH)
        def _():
            scatter_desc(j - (RING - DEPTH)).wait()

        @pl.when(slot == 0)
        def _():
            idx_desc(blk).wait()

            @pl.when(blk + 1 < NBLK)
            def _():
                idx_desc(blk + 1).start()

            for t in range(DEPTH):
                @pl.when(j + t < NCHUNK)
                def _():
                    gather_desc(j + t).start()

        @pl.when((slot + DEPTH < KB) & (j + DEPTH < NCHUNK))
        def _():
            gather_desc(j + DEPTH).start()

        gather_desc(j).wait()
        scatter_desc(j).start(add=True)

        b = lax.rem(blk, IBUF)
        for g in range(CHUNK // 16):
            sl = pl.ds(g * 16, 16)
            flat = (idx_v[b, slot, 1, sl] * 9
                    + idx_v[b, slot, 2, sl] * 3
                    + idx_v[b, slot, 3, sl]) - cnt_lo
            mask = (flat >= 0) & (flat < CNT_SPAN)
            flat = jnp.where(mask, flat, 0)
            plsc.addupdate_scatter(cnt_v, [flat], ones, mask=mask)

        return carry

    lax.fori_loop(0, NCHUNK, body, 0)

    for t in range(NCHUNK - (RING - DEPTH), NCHUNK):
        scatter_desc(t).wait()

    pltpu.sync_copy(cnt_v, cnt_out.at[c, s])
    plsc.subcore_barrier()

    pltpu.sync_copy(agg_sh.at[pl.ds(s * AGG_PER_TILE, AGG_PER_TILE)],
                    agg_out.at[c, pl.ds(s * AGG_PER_TILE, AGG_PER_TILE)])


_sc_call = pl.kernel(
    _sc_body,
    out_type=(
        jax.ShapeDtypeStruct((NC, NPAD, QW), jnp.float32),
        jax.ShapeDtypeStruct((NC, NS, CNT_WORDS), jnp.float32),
    ),
    mesh=plsc.VectorSubcoreMesh(core_axis_name="c", subcore_axis_name="s"),
    compiler_params=pltpu.CompilerParams(needs_layout_passes=False,
                                         use_tc_tiling_on_sc=False),
    scratch_types=[
        pltpu.VMEM((IBUF, KB, 4, CHUNK), jnp.int32),
        pltpu.VMEM((RING, CHUNK, QW), jnp.float32),
        pltpu.VMEM((CNT_WORDS,), jnp.float32),
        pltpu.SemaphoreType.DMA,
        pltpu.SemaphoreType.DMA,
        pltpu.SemaphoreType.DMA,
        pltpu.VMEM_SHARED((NPAD, QW), jnp.float32),
    ],
)


def _tcsum_body(c_ref, o_ref):
    o_ref[...] = jnp.sum(c_ref[...], axis=1)


def _tcsum(cnt_parts):
    blk = CNT_WORDS // 5
    return pl.pallas_call(
        _tcsum_body,
        grid=(5,),
        in_specs=[pl.BlockSpec((NC, NS, blk), lambda i: (0, 0, i))],
        out_specs=pl.BlockSpec((NC, blk), lambda i: (0, i)),
        out_shape=jax.ShapeDtypeStruct((NC, CNT_WORDS), jnp.float32),
    )(cnt_parts)


def _tc2_body(agg_ref, h_ref, cnt_ref, comb_ref, self_ref,
              w1_ref, b1_ref, w2_ref, b2_ref, o_ref):
    a = (jnp.concatenate([agg_ref[q] for q in range(NC)], axis=1)
         + jnp.concatenate([h_ref[q] for q in range(NC)], axis=1)
         + self_ref[...])
    a = a + lax.dot_general(cnt_ref[...], comb_ref[...],
                            (((1,), (0,)), ((), ())),
                            preferred_element_type=jnp.float32)
    z = lax.dot_general(a, w1_ref[...], (((1,), (1,)), ((), ())),
                        preferred_element_type=jnp.float32) + b1_ref[...]
    z = jnp.maximum(z, 0.0)
    o_ref[...] = lax.dot_general(z, w2_ref[...], (((1,), (1,)), ((), ())),
                                 preferred_element_type=jnp.float32) + b2_ref[...]


def _tc2(agg_parts, h2, cnt, comb, selfvec, W1, b1, W2, b2):
    blk = N // 10
    return pl.pallas_call(
        _tc2_body,
        grid=(10,),
        in_specs=[
            pl.BlockSpec((NC, blk, QW), lambda i: (0, i, 0)),
            pl.BlockSpec((NC, blk, QW), lambda i: (0, i, 0)),
            pl.BlockSpec((blk, 9), lambda i: (i, 0)),
            pl.BlockSpec((9, HID), lambda i: (0, 0)),
            pl.BlockSpec((1, HID), lambda i: (0, 0)),
            pl.BlockSpec((2 * HID, HID), lambda i: (0, 0)),
            pl.BlockSpec((1, 2 * HID), lambda i: (0, 0)),
            pl.BlockSpec((OUT, 2 * HID), lambda i: (0, 0)),
            pl.BlockSpec((1, OUT), lambda i: (0, 0)),
        ],
        out_specs=pl.BlockSpec((blk, OUT), lambda i: (i, 0)),
        out_shape=jax.ShapeDtypeStruct((N, OUT), jnp.float32),
    )(agg_parts, h2, cnt, comb, selfvec, W1, b1, W2, b2)


def kernel(x, edge_index, edge_attr, prelu_a, W, W1, b1, W2, b2, emb1, emb2):
    a_row = jnp.broadcast_to(prelu_a, (1, HID)).astype(jnp.float32)
    h2 = _tc1(x, a_row, W)

    pack = jnp.concatenate([edge_index, edge_attr.T], axis=0)
    npadrow = EPAD - E
    zpad = jnp.zeros((npadrow,), jnp.int32)
    pad_dst = N + (jnp.arange(npadrow, dtype=jnp.int32) % (NPAD - N))
    pack = jnp.concatenate(
        [pack, jnp.stack([zpad, pad_dst, zpad, zpad])], axis=1)
    pack = pack.reshape(4, NS, NCHUNK, CHUNK).transpose(1, 2, 0, 3)

    agg_parts, cnt_parts = _sc_call(h2, pack)

    csum = _tcsum(cnt_parts)
    cnt = jnp.concatenate(
        [csum[0, :CNT_SPAN], csum[1, :CNT_SPAN]]).reshape(N, 9)
    comb = (emb1[:3, None, :] + emb2[None, :3, :]).reshape(9, HID)
    selfvec = (emb1[4] + emb2[0]).reshape(1, HID)

    return _tc2(agg_parts, h2, cnt, comb, selfvec,
                W1, b1.reshape(1, -1), W2, b2.reshape(1, -1))


# speedup vs baseline: 1.0207x; 1.0164x over previous
import jax
import jax.numpy as jnp
from jax import lax
from jax.experimental import pallas as pl
from jax.experimental.pallas import tpu as pltpu
from jax.experimental.pallas import tpu_sc as plsc

N = 10000
E = 320000
HID = 128
OUT = 128

NC = 2
NS = 16
QW = HID // NC

CHUNK = 64
NCHUNK = 320
EPAD = NS * NCHUNK * CHUNK
KB = 16
NBLK = NCHUNK // KB
IBUF = 2
RING = 4
DEPTH = 2

NPAD = 10240
AGG_PER_TILE = NPAD // NS

NHALF = N // NC
CNT_SPAN = NHALF * 9
CNT_WORDS = 46080


def _tc1_body(x_ref, a_ref, w_ref, o_ref):
    xb = x_ref[...]
    hb = jnp.where(xb >= 0, xb, xb * a_ref[...])
    hb = lax.dot_general(hb, w_ref[...], (((1,), (1,)), ((), ())),
                         preferred_element_type=jnp.float32)
    for q in range(NC):
        o_ref[q] = hb[:, q * QW:(q + 1) * QW]


def _tc1(x, a_row, W):
    blk = N // 10
    return pl.pallas_call(
        _tc1_body,
        grid=(10,),
        in_specs=[
            pl.BlockSpec((blk, HID), lambda i: (i, 0)),
            pl.BlockSpec((1, HID), lambda i: (0, 0)),
            pl.BlockSpec((HID, HID), lambda i: (0, 0)),
        ],
        out_specs=pl.BlockSpec((NC, blk, QW), lambda i: (0, i, 0)),
        out_shape=jax.ShapeDtypeStruct((NC, N, QW), jnp.float32),
    )(x, a_row, W)


def _sc_body(h2_hbm, pack_hbm,
             agg_out, cnt_out,
             idx_v, rows_v, cnt_v, sem_i, sem_g, sem_s,
             agg_sh):
    c = lax.axis_index("c")
    s = lax.axis_index("s")

    z16 = jnp.zeros((16,), jnp.float32)
    ones = jnp.ones((16,), jnp.float32)
    cnt_lo = c * CNT_SPAN

    def zrow(i, carry):
        for g in range(QW // 16):
            rows_v[0, i, pl.ds(g * 16, 16)] = z16
        return carry
    lax.fori_loop(0, CHUNK, zrow, 0)

    def zcnt(i, carry):
        for g in range(8):
            cnt_v[pl.ds(i * 128 + g * 16, 16)] = z16
        return carry
    lax.fori_loop(0, CNT_WORDS // 128, zcnt, 0)

    for r in range(AGG_PER_TILE // CHUNK):
        pltpu.sync_copy(
            rows_v.at[0],
            agg_sh.at[pl.ds(s * AGG_PER_TILE + r * CHUNK, CHUNK)])
    plsc.subcore_barrier()

    def gather_desc(j):
        b = lax.rem(lax.div(j, KB), IBUF)
        return pltpu.make_async_copy(
            h2_hbm.at[c].at[idx_v.at[b, lax.rem(j, KB), 0]],
            rows_v.at[lax.rem(j, RING)], sem_g)

    def scatter_desc(j):
        b = lax.rem(lax.div(j, KB), IBUF)
        return pltpu.make_async_copy(
            rows_v.at[lax.rem(j, RING)],
            agg_sh.at[idx_v.at[b, lax.rem(j, KB), 1]], sem_s)

    def idx_desc(blk):
        return pltpu.make_async_copy(
            pack_hbm.at[s, pl.ds(blk * KB, KB)],
            idx_v.at[lax.rem(blk, IBUF)], sem_i)

    idx_desc(0).start()

    def body(j, carry):
        slot = lax.rem(j, KB)
        blk = lax.div(j, KB)

        @pl.when(j >= RING - DEPTH)
        def _():
            scatter_desc(j - (RING - DEPTH)).wait()

        @pl.when(slot == 0)
        def _():
            idx_desc(blk).wait()

            @pl.when(blk + 1 < NBLK)
            def _():
                idx_desc(blk + 1).start()

            for t in range(DEPTH):
                @pl.when(j + t < NCHUNK)
                def _():
                    gather_desc(j + t).start()

        @pl.when((slot + DEPTH < KB) & (j + DEPTH < NCHUNK))
        def _():
            gather_desc(j + DEPTH).start()

        gather_desc(j).wait()
        scatter_desc(j).start(add=True)

        b = lax.rem(blk, IBUF)
        for g in range(CHUNK // 16):
            sl = pl.ds(g * 16, 16)
            flat = (idx_v[b, slot, 1, sl] * 9
                    + idx_v[b, slot, 2, sl] * 3
                    + idx_v[b, slot, 3, sl]) - cnt_lo
            mask = (flat >= 0) & (flat < CNT_SPAN)
            flat = jnp.where(mask, flat, 0)
            plsc.addupdate_scatter(cnt_v, [flat], ones, mask=mask)

        return carry

    lax.fori_loop(0, NCHUNK, body, 0)

    for t in range(NCHUNK - (RING - DEPTH), NCHUNK):
        scatter_desc(t).wait()

    pltpu.sync_copy(cnt_v, cnt_out.at[c, s])
    plsc.subcore_barrier()

    pltpu.sync_copy(agg_sh.at[pl.ds(s * AGG_PER_TILE, AGG_PER_TILE)],
                    agg_out.at[c, pl.ds(s * AGG_PER_TILE, AGG_PER_TILE)])


_sc_call = pl.kernel(
    _sc_body,
    out_type=(
        jax.ShapeDtypeStruct((NC, NPAD, QW), jnp.float32),
        jax.ShapeDtypeStruct((NC, NS, CNT_WORDS), jnp.float32),
    ),
    mesh=plsc.VectorSubcoreMesh(core_axis_name="c", subcore_axis_name="s"),
    compiler_params=pltpu.CompilerParams(needs_layout_passes=False,
                                         use_tc_tiling_on_sc=False),
    scratch_types=[
        pltpu.VMEM((IBUF, KB, 4, CHUNK), jnp.int32),
        pltpu.VMEM((RING, CHUNK, QW), jnp.float32),
        pltpu.VMEM((CNT_WORDS,), jnp.float32),
        pltpu.SemaphoreType.DMA,
        pltpu.SemaphoreType.DMA,
        pltpu.SemaphoreType.DMA,
        pltpu.VMEM_SHARED((NPAD, QW), jnp.float32),
    ],
)


def _tcsum_body(c_ref, o_ref):
    o_ref[...] = jnp.sum(c_ref[...], axis=1)


def _tcsum(cnt_parts):
    blk = CNT_WORDS // 5
    return pl.pallas_call(
        _tcsum_body,
        grid=(5,),
        in_specs=[pl.BlockSpec((NC, NS, blk), lambda i: (0, 0, i))],
        out_specs=pl.BlockSpec((NC, blk), lambda i: (0, i)),
        out_shape=jax.ShapeDtypeStruct((NC, CNT_WORDS), jnp.float32),
    )(cnt_parts)


def _tc2_body(agg_ref, h_ref, cnt_ref, comb_ref, self_ref,
              w1_ref, b1_ref, w2_ref, b2_ref, o_ref):
    a = (jnp.concatenate([agg_ref[q] for q in range(NC)], axis=1)
         + jnp.concatenate([h_ref[q] for q in range(NC)], axis=1)
         + self_ref[...])
    a = a + lax.dot_general(cnt_ref[...], comb_ref[...],
                            (((1,), (0,)), ((), ())),
                            preferred_element_type=jnp.float32)
    z = lax.dot_general(a, w1_ref[...], (((1,), (1,)), ((), ())),
                        preferred_element_type=jnp.float32) + b1_ref[...]
    z = jnp.maximum(z, 0.0)
    o_ref[...] = lax.dot_general(z, w2_ref[...], (((1,), (1,)), ((), ())),
                                 preferred_element_type=jnp.float32) + b2_ref[...]


def _tc2(agg_parts, h2, cnt, comb, selfvec, W1, b1, W2, b2):
    blk = N // 10
    return pl.pallas_call(
        _tc2_body,
        grid=(10,),
        in_specs=[
            pl.BlockSpec((NC, blk, QW), lambda i: (0, i, 0)),
            pl.BlockSpec((NC, blk, QW), lambda i: (0, i, 0)),
            pl.BlockSpec((blk, 9), lambda i: (i, 0)),
            pl.BlockSpec((9, HID), lambda i: (0, 0)),
            pl.BlockSpec((1, HID), lambda i: (0, 0)),
            pl.BlockSpec((2 * HID, HID), lambda i: (0, 0)),
            pl.BlockSpec((1, 2 * HID), lambda i: (0, 0)),
            pl.BlockSpec((OUT, 2 * HID), lambda i: (0, 0)),
            pl.BlockSpec((1, OUT), lambda i: (0, 0)),
        ],
        out_specs=pl.BlockSpec((blk, OUT), lambda i: (i, 0)),
        out_shape=jax.ShapeDtypeStruct((N, OUT), jnp.float32),
    )(agg_parts, h2, cnt, comb, selfvec, W1, b1, W2, b2)


def kernel(x, edge_index, edge_attr, prelu_a, W, W1, b1, W2, b2, emb1, emb2):
    a_row = jnp.broadcast_to(prelu_a, (1, HID)).astype(jnp.float32)
    h2 = _tc1(x, a_row, W)

    pack = jnp.concatenate([edge_index, edge_attr.T], axis=0)
    padcol = jnp.array([0, NPAD - 1, 0, 0], jnp.int32)[:, None]
    pack = jnp.concatenate(
        [pack, jnp.broadcast_to(padcol, (4, EPAD - E))], axis=1)
    pack = pack.reshape(4, NS, NCHUNK, CHUNK).transpose(1, 2, 0, 3)

    agg_parts, cnt_parts = _sc_call(h2, pack)

    csum = _tcsum(cnt_parts)
    cnt = jnp.concatenate(
        [csum[0, :CNT_SPAN], csum[1, :CNT_SPAN]]).reshape(N, 9)
    comb = (emb1[:3, None, :] + emb2[None, :3, :]).reshape(9, HID)
    selfvec = (emb1[4] + emb2[0]).reshape(1, HID)

    return _tc2(agg_parts, h2, cnt, comb, selfvec,
                W1, b1.reshape(1, -1), W2, b2.reshape(1, -1))
